# baseline (speedup 1.0000x reference)
import functools

import jax
import jax.numpy as jnp
from jax import lax
from jax.experimental import pallas as pl
from jax.experimental.pallas import tpu as pltpu
from jax.experimental.pallas import tpu_sc as plsc

SEQ = 2048
D = 16
N = SEQ * SEQ
NC, NS = 2, 16
NW = NC * NS
L = 16
ROW_OUT = 2 * 16 * 8 * 128
IBLK = 16 * 8 * 128
BLKS_PER_W = (SEQ // 8) // NW

_mesh = plsc.VectorSubcoreMesh(core_axis_name="c", subcore_axis_name="s")


@functools.partial(
    pl.kernel,
    out_type=jax.ShapeDtypeStruct((SEQ * ROW_OUT,), jnp.float32),
    mesh=_mesh,
    compiler_params=pltpu.CompilerParams(
        use_tc_tiling_on_sc=False, needs_layout_passes=False),
    scratch_types=[
        pltpu.VMEM((2, IBLK), jnp.int32),
        pltpu.VMEM((2, ROW_OUT), jnp.float32),
        pltpu.VMEM((D, 129), jnp.float32),
        pltpu.SemaphoreType.DMA,
        pltpu.SemaphoreType.DMA,
        pltpu.SemaphoreType.DMA,
        pltpu.SemaphoreType.DMA,
    ],
)
def _srpe_gather(idx_hbm, tabt_hbm, out_hbm, idx_v, row_v, tabt_v,
                 sem_i0, sem_i1, sem_o0, sem_o1):
    wid = lax.axis_index("s") * NC + lax.axis_index("c")
    first_blk = wid * BLKS_PER_W

    pltpu.sync_copy(tabt_hbm, tabt_v)

    sem_i = (sem_i0, sem_i1)
    sem_o = (sem_o0, sem_o1)

    def idx_src(blk):
        off = pl.multiple_of(blk * IBLK, IBLK)
        return idx_hbm.at[pl.ds(off, IBLK)]

    def out_dst(row):
        off = pl.multiple_of(row * ROW_OUT, ROW_OUT)
        return out_hbm.at[pl.ds(off, ROW_OUT)]

    def gather_row(ib, i_in, rb):
        ibuf = idx_v.at[ib]
        obuf = row_v.at[rb]

        def _group(k, carry):
            jb = k >> 3
            g = k & 7
            io = jb * 1024 + i_in * 128 + g * L
            oo = jb * 1024 + g * L
            idx_vec = ibuf[pl.ds(io, L)]
            for d in range(D):
                vals = plsc.load_gather(tabt_v.at[d], [idx_vec])
                obuf[pl.ds(oo + (d // 8) * 16384 + (d % 8) * 128, L)] = vals
            return carry

        lax.fori_loop(0, 128, _group, 0)

    pltpu.async_copy(idx_src(first_blk), idx_v.at[0], sem_i[0])

    def blk_body(b2, carry):
        for ib in range(2):
            b = b2 * 2 + ib
            blk = first_blk + b

            @pl.when(b + 1 < BLKS_PER_W)
            def _prefetch():
                pltpu.async_copy(
                    idx_src(blk + 1), idx_v.at[1 - ib], sem_i[1 - ib])

            pltpu.make_async_copy(idx_src(blk), idx_v.at[ib], sem_i[ib]).wait()

            def iin_body(ii, carry2, b=b, blk=blk, ib=ib):
                for rb in range(2):
                    i_in = ii * 2 + rb
                    row = blk * 8 + i_in

                    @pl.when(b * 8 + i_in >= 2)
                    def _drain():
                        pltpu.make_async_copy(
                            row_v.at[rb], out_dst(row), sem_o[rb]).wait()

                    gather_row(ib, i_in, rb)
                    pltpu.async_copy(row_v.at[rb], out_dst(row), sem_o[rb])
                return carry2

            lax.fori_loop(0, 4, iin_body, 0)
        return carry

    lax.fori_loop(0, BLKS_PER_W // 2, blk_body, 0)

    last = (first_blk + BLKS_PER_W) * 8
    pltpu.make_async_copy(row_v.at[0], out_dst(last - 2), sem_o[0]).wait()
    pltpu.make_async_copy(row_v.at[1], out_dst(last - 1), sem_o[1]).wait()


def kernel(SDist, srpe_weight):
    idx = SDist.reshape(SEQ // 8, 8, SEQ // 128, 128).transpose(0, 2, 1, 3)
    idx = idx.reshape(N)
    tabt = srpe_weight.T
    flat = _srpe_gather(idx, tabt)
    out = flat.reshape(SEQ, 2, 16, 8, 128).transpose(0, 2, 4, 1, 3)
    return out.reshape(SEQ, SEQ, D)


# speedup vs baseline: 2.8458x; 2.8458x over previous
import functools

import jax
import jax.numpy as jnp
from jax import lax
from jax.experimental import pallas as pl
from jax.experimental.pallas import tpu as pltpu
from jax.experimental.pallas import tpu_sc as plsc

SEQ = 2048
D = 16
N = SEQ * SEQ
NC, NS = 2, 16
NW = NC * NS
L = 16
ROW_OUT = 2 * 16 * 8 * 128
IBLK = 16 * 8 * 128
BLKS_PER_W = (SEQ // 8) // NW

_mesh = plsc.VectorSubcoreMesh(core_axis_name="c", subcore_axis_name="s")


@functools.partial(
    pl.kernel,
    out_type=jax.ShapeDtypeStruct((SEQ * ROW_OUT,), jnp.float32),
    mesh=_mesh,
    compiler_params=pltpu.CompilerParams(
        use_tc_tiling_on_sc=False, needs_layout_passes=False),
    scratch_types=[
        pltpu.VMEM((2, IBLK), jnp.int32),
        pltpu.VMEM((2, ROW_OUT), jnp.float32),
        pltpu.VMEM((D, 129), jnp.float32),
        pltpu.SemaphoreType.DMA,
        pltpu.SemaphoreType.DMA,
        pltpu.SemaphoreType.DMA,
        pltpu.SemaphoreType.DMA,
    ],
)
def _srpe_gather(idx_hbm, tabt_hbm, out_hbm, idx_v, row_v, tabt_v,
                 sem_i0, sem_i1, sem_o0, sem_o1):
    wid = lax.axis_index("s") * NC + lax.axis_index("c")
    first_blk = wid * BLKS_PER_W

    pltpu.sync_copy(tabt_hbm, tabt_v)

    sem_i = (sem_i0, sem_i1)
    sem_o = (sem_o0, sem_o1)

    def idx_src(blk):
        off = pl.multiple_of(blk * IBLK, IBLK)
        return idx_hbm.at[pl.ds(off, IBLK)]

    def out_dst(row):
        off = pl.multiple_of(row * ROW_OUT, ROW_OUT)
        return out_hbm.at[pl.ds(off, ROW_OUT)]

    def gather_row(ib, i_in, rb):
        ibuf = idx_v.at[ib]
        obuf = row_v.at[rb]

        def _group(k, carry):
            jb = k >> 3
            g = k & 7
            io = jb * 1024 + i_in * 128 + g * L
            oo = jb * 1024 + g * L
            idx_vec = ibuf[pl.ds(io, L)]
            vals = [plsc.load_gather(tabt_v.at[d], [idx_vec])
                    for d in range(D)]
            for d in range(D):
                obuf[pl.ds(oo + (d // 8) * 16384 + (d % 8) * 128, L)] = vals[d]
            return carry

        lax.fori_loop(0, 128, _group, 0)

    pltpu.async_copy(idx_src(first_blk), idx_v.at[0], sem_i[0])

    def blk_body(b2, carry):
        for ib in range(2):
            b = b2 * 2 + ib
            blk = first_blk + b

            @pl.when(b + 1 < BLKS_PER_W)
            def _prefetch():
                pltpu.async_copy(
                    idx_src(blk + 1), idx_v.at[1 - ib], sem_i[1 - ib])

            pltpu.make_async_copy(idx_src(blk), idx_v.at[ib], sem_i[ib]).wait()

            def iin_body(ii, carry2, b=b, blk=blk, ib=ib):
                for rb in range(2):
                    i_in = ii * 2 + rb
                    row = blk * 8 + i_in

                    @pl.when(b * 8 + i_in >= 2)
                    def _drain():
                        pltpu.make_async_copy(
                            row_v.at[rb], out_dst(row), sem_o[rb]).wait()

                    gather_row(ib, i_in, rb)
                    pltpu.async_copy(row_v.at[rb], out_dst(row), sem_o[rb])
                return carry2

            lax.fori_loop(0, 4, iin_body, 0)
        return carry

    lax.fori_loop(0, BLKS_PER_W // 2, blk_body, 0)

    last = (first_blk + BLKS_PER_W) * 8
    pltpu.make_async_copy(row_v.at[0], out_dst(last - 2), sem_o[0]).wait()
    pltpu.make_async_copy(row_v.at[1], out_dst(last - 1), sem_o[1]).wait()


def kernel(SDist, srpe_weight):
    idx = SDist.reshape(SEQ // 8, 8, SEQ // 128, 128).transpose(0, 2, 1, 3)
    idx = idx.reshape(N)
    tabt = srpe_weight.T
    flat = _srpe_gather(idx, tabt)
    out = flat.reshape(SEQ, 2, 16, 8, 128).transpose(0, 2, 4, 1, 3)
    return out.reshape(SEQ, SEQ, D)


# speedup vs baseline: 3.0688x; 1.0784x over previous
import functools

import jax
import jax.numpy as jnp
from jax import lax
from jax.experimental import pallas as pl
from jax.experimental.pallas import tpu as pltpu
from jax.experimental.pallas import tpu_sc as plsc

SEQ = 2048
D = 16
N = SEQ * SEQ
NC, NS = 2, 16
NW = NC * NS
L = 16
ROW_OUT = 2 * 16 * 8 * 128
IBLK = 16 * 8 * 128
BLKS_PER_W = (SEQ // 8) // NW

_mesh = plsc.VectorSubcoreMesh(core_axis_name="c", subcore_axis_name="s")


@functools.partial(
    pl.kernel,
    out_type=jax.ShapeDtypeStruct((SEQ * ROW_OUT,), jnp.float32),
    mesh=_mesh,
    compiler_params=pltpu.CompilerParams(
        use_tc_tiling_on_sc=False, needs_layout_passes=False),
    scratch_types=[
        pltpu.VMEM((2, IBLK), jnp.int32),
        pltpu.VMEM((2, ROW_OUT), jnp.float32),
        pltpu.VMEM((D, 129), jnp.float32),
        pltpu.SemaphoreType.DMA,
        pltpu.SemaphoreType.DMA,
        pltpu.SemaphoreType.DMA,
        pltpu.SemaphoreType.DMA,
    ],
)
def _srpe_gather(idx_hbm, tabt_hbm, out_hbm, idx_v, row_v, tabt_v,
                 sem_i0, sem_i1, sem_o0, sem_o1):
    wid = lax.axis_index("s") * NC + lax.axis_index("c")
    first_blk = wid * BLKS_PER_W

    pltpu.sync_copy(tabt_hbm, tabt_v)

    sem_i = (sem_i0, sem_i1)
    sem_o = (sem_o0, sem_o1)

    def idx_src(blk):
        off = pl.multiple_of(blk * IBLK, IBLK)
        return idx_hbm.at[pl.ds(off, IBLK)]

    def out_dst(row):
        off = pl.multiple_of(row * ROW_OUT, ROW_OUT)
        return out_hbm.at[pl.ds(off, ROW_OUT)]

    def gather_row(ib, i_in, rb):
        ibuf = idx_v.at[ib]
        obuf = row_v.at[rb]

        def _group(t, carry):
            oos, vals = [], []
            for u in range(2):
                k = t * 2 + u
                jb = k >> 3
                g = k & 7
                io = jb * 1024 + i_in * 128 + g * L
                oos.append(jb * 1024 + g * L)
                idx_vec = ibuf[pl.ds(io, L)]
                vals.append([plsc.load_gather(tabt_v.at[d], [idx_vec])
                             for d in range(D)])
            for u in range(2):
                for d in range(D):
                    obuf[pl.ds(oos[u] + (d // 8) * 16384 + (d % 8) * 128,
                               L)] = vals[u][d]
            return carry

        lax.fori_loop(0, 64, _group, 0)

    pltpu.async_copy(idx_src(first_blk), idx_v.at[0], sem_i[0])

    def blk_body(b2, carry):
        for ib in range(2):
            b = b2 * 2 + ib
            blk = first_blk + b

            @pl.when(b + 1 < BLKS_PER_W)
            def _prefetch():
                pltpu.async_copy(
                    idx_src(blk + 1), idx_v.at[1 - ib], sem_i[1 - ib])

            pltpu.make_async_copy(idx_src(blk), idx_v.at[ib], sem_i[ib]).wait()

            def iin_body(ii, carry2, b=b, blk=blk, ib=ib):
                for rb in range(2):
                    i_in = ii * 2 + rb
                    row = blk * 8 + i_in

                    @pl.when(b * 8 + i_in >= 2)
                    def _drain():
                        pltpu.make_async_copy(
                            row_v.at[rb], out_dst(row), sem_o[rb]).wait()

                    gather_row(ib, i_in, rb)
                    pltpu.async_copy(row_v.at[rb], out_dst(row), sem_o[rb])
                return carry2

            lax.fori_loop(0, 4, iin_body, 0)
        return carry

    lax.fori_loop(0, BLKS_PER_W // 2, blk_body, 0)

    last = (first_blk + BLKS_PER_W) * 8
    pltpu.make_async_copy(row_v.at[0], out_dst(last - 2), sem_o[0]).wait()
    pltpu.make_async_copy(row_v.at[1], out_dst(last - 1), sem_o[1]).wait()


def kernel(SDist, srpe_weight):
    idx = SDist.reshape(SEQ // 8, 8, SEQ // 128, 128).transpose(0, 2, 1, 3)
    idx = idx.reshape(N)
    tabt = srpe_weight.T
    flat = _srpe_gather(idx, tabt)
    out = flat.reshape(SEQ, 2, 16, 8, 128).transpose(0, 2, 4, 1, 3)
    return out.reshape(SEQ, SEQ, D)
